# Initial kernel scaffold; baseline (speedup 1.0000x reference)
import jax
import jax.numpy as jnp
from jax.experimental import pallas as pl


def kernel(x, edge_index, W1_l, W1_r, b1, W2_l, W2_r, b2):
    raise NotImplementedError("write your pallas kernel here")


# speedup vs baseline: 4.7473x; 4.7473x over previous
import functools

import jax
import jax.numpy as jnp
from jax import lax
from jax.experimental import pallas as pl
from jax.experimental.pallas import tpu as pltpu
from jax.experimental.pallas import tpu_sc as plsc

N = 10000
E = 160000
IN_CH = 256
HID = 512
OUT_CH = 256

NC, NS, L = 2, 16, 16
CH = 128
EPT = E // NS
K = 80
NCHUNK = EPT // K
NPT = N // NS
ZR = 125
NZ = NPT // ZR


def _sc_seg_body(xa, xb, srcr, dstr, agg_a, agg_b, deg_out,
                 acc, degacc, zbuf, zdeg, rows, srcv, dstv, onesv):
    c = lax.axis_index("c")
    s = lax.axis_index("s")
    zvec = jnp.zeros((L,), jnp.float32)

    @pl.loop(0, ZR)
    def _(i):
        for j in range(CH // L):
            zbuf[i, pl.ds(j * L, L)] = zvec
        zdeg[i, :] = zvec

    @pl.loop(0, K)
    def _(i):
        onesv[i, :] = jnp.ones((L,), jnp.float32)

    n0 = s * NPT

    @pl.loop(0, NZ)
    def _(k):
        pltpu.sync_copy(zbuf, acc.at[pl.ds(n0 + k * ZR, ZR)])

    @pl.loop(0, NZ)
    def _(k):
        pltpu.sync_copy(zdeg, degacc.at[pl.ds(n0 + k * ZR, ZR)])

    plsc.subcore_barrier()

    e0 = s * EPT

    def edge_loop(feats):
        @pl.loop(0, NCHUNK)
        def _(g):
            base = e0 + g * K
            pltpu.sync_copy(srcr.at[pl.ds(base, K)], srcv)
            pltpu.sync_copy(dstr.at[pl.ds(base, K)], dstv)
            pltpu.sync_copy(feats.at[srcv], rows)
            pltpu.sync_copy(rows, acc.at[dstv], add=True)
            pltpu.sync_copy(onesv, degacc.at[dstv], add=True)

    @pl.when(c == 0)
    def _():
        edge_loop(xa)

    @pl.when(c == 1)
    def _():
        edge_loop(xb)

    plsc.subcore_barrier()

    def write_agg(out):
        @pl.loop(0, NZ)
        def _(k):
            r0 = n0 + k * ZR
            pltpu.sync_copy(acc.at[pl.ds(r0, ZR)], zbuf)
            pltpu.sync_copy(zbuf, out.at[pl.ds(r0, ZR)])

    @pl.when(c == 0)
    def _():
        write_agg(agg_a)

        @pl.loop(0, NZ)
        def _(k):
            r0 = n0 + k * ZR
            pltpu.sync_copy(degacc.at[pl.ds(r0, ZR)], zdeg)
            pltpu.sync_copy(zdeg, deg_out.at[pl.ds(r0, ZR)])

    @pl.when(c == 1)
    def _():
        write_agg(agg_b)


@functools.cache
def _get_sc_seg():
    return pl.kernel(
        _sc_seg_body,
        out_type=(
            jax.ShapeDtypeStruct((N, CH), jnp.float32),
            jax.ShapeDtypeStruct((N, CH), jnp.float32),
            jax.ShapeDtypeStruct((N, L), jnp.float32),
        ),
        mesh=plsc.VectorSubcoreMesh(core_axis_name="c", subcore_axis_name="s",
                                    num_cores=NC, num_subcores=NS),
        compiler_params=pltpu.CompilerParams(use_tc_tiling_on_sc=False),
        scratch_types=(
        pltpu.VMEM_SHARED((N, CH), jnp.float32),
        pltpu.VMEM_SHARED((N, L), jnp.float32),
        pltpu.VMEM((ZR, CH), jnp.float32),
        pltpu.VMEM((ZR, L), jnp.float32),
        pltpu.VMEM((K, CH), jnp.float32),
        pltpu.VMEM((K,), jnp.int32),
        pltpu.VMEM((K,), jnp.int32),
        pltpu.VMEM((K, L), jnp.float32),
        ),
    )


M_BLK = 2000


def _tc1_body(agga, aggb, deg, x, w1l, w1r, b1, w2l, w2r, za, zb, y):
    r = 1.0 / jnp.maximum(deg[:, 0:1], 1.0)
    ma = agga[:, :] * r
    mb = aggb[:, :] * r
    acc = jnp.dot(ma, w1l[0:CH, :], preferred_element_type=jnp.float32)
    acc = acc + jnp.dot(mb, w1l[CH:2 * CH, :], preferred_element_type=jnp.float32)
    acc = acc + jnp.dot(x[:, :], w1r[:, :], preferred_element_type=jnp.float32)
    h = jnp.maximum(acc + b1[:, :], 0.0)
    za[:, :] = jnp.dot(h, w2l[:, 0:CH], preferred_element_type=jnp.float32)
    zb[:, :] = jnp.dot(h, w2l[:, CH:2 * CH], preferred_element_type=jnp.float32)
    y[:, :] = jnp.dot(h, w2r[:, :], preferred_element_type=jnp.float32)


_tc1 = pl.pallas_call(
    _tc1_body,
    grid=(N // M_BLK,),
    in_specs=[
        pl.BlockSpec((M_BLK, CH), lambda i: (i, 0)),
        pl.BlockSpec((M_BLK, CH), lambda i: (i, 0)),
        pl.BlockSpec((M_BLK, L), lambda i: (i, 0)),
        pl.BlockSpec((M_BLK, IN_CH), lambda i: (i, 0)),
        pl.BlockSpec((IN_CH, HID), lambda i: (0, 0)),
        pl.BlockSpec((IN_CH, HID), lambda i: (0, 0)),
        pl.BlockSpec((1, HID), lambda i: (0, 0)),
        pl.BlockSpec((HID, OUT_CH), lambda i: (0, 0)),
        pl.BlockSpec((HID, OUT_CH), lambda i: (0, 0)),
    ],
    out_specs=[
        pl.BlockSpec((M_BLK, CH), lambda i: (i, 0)),
        pl.BlockSpec((M_BLK, CH), lambda i: (i, 0)),
        pl.BlockSpec((M_BLK, OUT_CH), lambda i: (i, 0)),
    ],
    out_shape=[
        jax.ShapeDtypeStruct((N, CH), jnp.float32),
        jax.ShapeDtypeStruct((N, CH), jnp.float32),
        jax.ShapeDtypeStruct((N, OUT_CH), jnp.float32),
    ],
)


def _tc2_body(agga, aggb, deg, y, b2, out):
    r = 1.0 / jnp.maximum(deg[:, 0:1], 1.0)
    m = jnp.concatenate([agga[:, :] * r, aggb[:, :] * r], axis=1)
    out[:, :] = jax.nn.sigmoid(m + y[:, :] + b2[:, :])


_tc2 = pl.pallas_call(
    _tc2_body,
    grid=(N // M_BLK,),
    in_specs=[
        pl.BlockSpec((M_BLK, CH), lambda i: (i, 0)),
        pl.BlockSpec((M_BLK, CH), lambda i: (i, 0)),
        pl.BlockSpec((M_BLK, L), lambda i: (i, 0)),
        pl.BlockSpec((M_BLK, OUT_CH), lambda i: (i, 0)),
        pl.BlockSpec((1, OUT_CH), lambda i: (0, 0)),
    ],
    out_specs=pl.BlockSpec((M_BLK, OUT_CH), lambda i: (i, 0)),
    out_shape=jax.ShapeDtypeStruct((N, OUT_CH), jnp.float32),
)


def kernel(x, edge_index, W1_l, W1_r, b1, W2_l, W2_r, b2):
    src = edge_index[0].astype(jnp.int32)
    dst = edge_index[1].astype(jnp.int32)
    xa = x[:, :CH]
    xb = x[:, CH:]
    _sc_seg = _get_sc_seg()
    agg_a, agg_b, deg = _sc_seg(xa, xb, src, dst)
    za, zb, y = _tc1(agg_a, agg_b, deg, x, W1_l, W1_r,
                     b1.reshape(1, HID), W2_l, W2_r)
    ga, gb, _ = _sc_seg(za, zb, src, dst)
    return _tc2(ga, gb, deg, y, b2.reshape(1, OUT_CH))
